# baseline (speedup 1.0000x reference)
import functools

import jax
import jax.numpy as jnp
from jax import lax
from jax.experimental import pallas as pl
from jax.experimental.pallas import tpu as pltpu
from jax.experimental.pallas import tpu_sc as plsc

NUM_CORES = 2
NUM_SUBCORES = 16
NUM_WORKERS = NUM_CORES * NUM_SUBCORES

DIM = 64
CHUNK = 128
NBUF = 4


def _embed_kernel(seq, n_chunks, n_items, table_hbm, xt_hbm, out_hbm,
                  idx_v, rows_v, t_v, isems, gsems, osems):
  wid = lax.axis_index("s") * NUM_CORES + lax.axis_index("c")
  item0 = wid * n_items
  iota = lax.iota(jnp.int32, 16)

  def sb(i):
    t = item0 + i
    return t // n_chunks, (t % n_chunks) * CHUNK

  def fire_idx(b, i):
    s, b0 = sb(i)
    pltpu.async_copy(xt_hbm.at[s, pl.ds(b0, CHUNK)], idx_v.at[b], isems[b])

  def wait_idx(b, i):
    s, b0 = sb(i)
    pltpu.make_async_copy(
        xt_hbm.at[s, pl.ds(b0, CHUNK)], idx_v.at[b], isems[b]).wait()

  def fire_g(b):
    pltpu.async_copy(table_hbm.at[idx_v.at[b]], rows_v.at[b], gsems[b])

  def wait_g(b):
    pltpu.make_async_copy(
        table_hbm.at[idx_v.at[b]], rows_v.at[b], gsems[b]).wait()

  def transpose(b):
    @pl.loop(0, CHUNK // 16)
    def _grp(j):
      row_idx = j * 16 + iota
      for d in range(DIM):
        vec = plsc.load_gather(
            rows_v.at[b], [row_idx, jnp.full((16,), d, jnp.int32)])
        t_v[b, d, pl.ds(j * 16, 16)] = vec

  def fire_out(b, i):
    s, b0 = sb(i)
    pltpu.async_copy(t_v.at[b], out_hbm.at[s, :, pl.ds(b0, CHUNK)], osems[b])

  def wait_out(b, i):
    s, b0 = sb(i)
    pltpu.make_async_copy(
        t_v.at[b], out_hbm.at[s, :, pl.ds(b0, CHUNK)], osems[b]).wait()

  fire_idx(0, 0)
  fire_idx(1, 1)
  wait_idx(0, 0)
  fire_g(0)
  fire_idx(2, 2)
  wait_idx(1, 1)
  fire_g(1)
  fire_idx(3, 3)
  for b in range(NBUF):
    i = b
    wait_g(b)
    transpose(b)
    fire_out(b, i)
    fire_idx(b, i + NBUF)
    nb = (b + 2) % NBUF
    wait_idx(nb, i + 2)
    fire_g(nb)

  @pl.loop(NBUF, n_items - NBUF, step=NBUF)
  def _steady(g0):
    for b in range(NBUF):
      i = g0 + b
      wait_g(b)
      wait_out(b, i - NBUF)
      transpose(b)
      fire_out(b, i)
      fire_idx(b, i + NBUF)
      nb = (b + 2) % NBUF
      wait_idx(nb, i + 2)
      fire_g(nb)

  for b in range(NBUF):
    i = n_items - NBUF + b
    wait_g(b)
    wait_out(b, i - NBUF)
    transpose(b)
    fire_out(b, i)
    if b < 2:
      nb = (b + 2) % NBUF
      wait_idx(nb, i + 2)
      fire_g(nb)
  for b in range(NBUF):
    wait_out(b, n_items - NBUF + b)


def kernel(x, table):
  batch, seq = x.shape
  n_chunks = batch // CHUNK
  total_items = seq * n_chunks
  assert total_items % NUM_WORKERS == 0
  n_items = total_items // NUM_WORKERS
  assert n_items % NBUF == 0 and n_items >= 3 * NBUF

  xt = jnp.transpose(x.astype(jnp.int32))

  mesh = plsc.VectorSubcoreMesh(core_axis_name="c", subcore_axis_name="s")
  run = pl.kernel(
      functools.partial(_embed_kernel, seq, n_chunks, n_items),
      out_type=jax.ShapeDtypeStruct((seq, DIM, batch), jnp.float32),
      mesh=mesh,
      scratch_types=[
          pltpu.VMEM((NBUF, CHUNK), jnp.int32),
          pltpu.VMEM((NBUF, CHUNK, DIM), jnp.float32),
          pltpu.VMEM((NBUF, DIM, CHUNK), jnp.float32),
          [pltpu.SemaphoreType.DMA] * NBUF,
          [pltpu.SemaphoreType.DMA] * NBUF,
          [pltpu.SemaphoreType.DMA] * NBUF,
      ],
      compiler_params=pltpu.CompilerParams(
          use_tc_tiling_on_sc=False, needs_layout_passes=False),
  )
  out = run(table, xt)
  return jnp.transpose(out, (2, 0, 1))


# speedup vs baseline: 1.3227x; 1.3227x over previous
import functools

import jax
import jax.numpy as jnp
from jax import lax
from jax.experimental import pallas as pl
from jax.experimental.pallas import tpu as pltpu
from jax.experimental.pallas import tpu_sc as plsc

NUM_CORES = 2
NUM_SUBCORES = 16
NUM_WORKERS = NUM_CORES * NUM_SUBCORES

DIM = 64
CHUNK = 128
NBUF = 4


def _embed_kernel(seq, n_chunks, n_items, table_hbm, xt_hbm, out_hbm,
                  idx_v, rows_v, t_v, isems, gsems, osems):
  wid = lax.axis_index("s") * NUM_CORES + lax.axis_index("c")
  item0 = wid * n_items
  iota = lax.iota(jnp.int32, 16)

  def sb(i):
    t = item0 + i
    return t // n_chunks, (t % n_chunks) * CHUNK

  def fire_idx(b, i):
    s, b0 = sb(i)
    pltpu.async_copy(xt_hbm.at[s, pl.ds(b0, CHUNK)], idx_v.at[b], isems[b])

  def wait_idx(b, i):
    s, b0 = sb(i)
    pltpu.make_async_copy(
        xt_hbm.at[s, pl.ds(b0, CHUNK)], idx_v.at[b], isems[b]).wait()

  def fire_g(b):
    pltpu.async_copy(table_hbm.at[idx_v.at[b]], rows_v.at[b], gsems[b])

  def wait_g(b):
    pltpu.make_async_copy(
        table_hbm.at[idx_v.at[b]], rows_v.at[b], gsems[b]).wait()

  def transpose(b):
    @pl.loop(0, CHUNK // 16)
    def _grp(j):
      row_idx = j * 16 + iota

      @plsc.parallel_loop(0, DIM, unroll=16)
      def _t(d):
        vec = plsc.load_gather(
            rows_v.at[b], [row_idx, jnp.full((16,), d, jnp.int32)])
        t_v[b, d, pl.ds(j * 16, 16)] = vec

  def fire_out(b, i):
    s, b0 = sb(i)
    pltpu.async_copy(t_v.at[b], out_hbm.at[s, :, pl.ds(b0, CHUNK)], osems[b])

  def wait_out(b, i):
    s, b0 = sb(i)
    pltpu.make_async_copy(
        t_v.at[b], out_hbm.at[s, :, pl.ds(b0, CHUNK)], osems[b]).wait()

  fire_idx(0, 0)
  fire_idx(1, 1)
  wait_idx(0, 0)
  fire_g(0)
  fire_idx(2, 2)
  wait_idx(1, 1)
  fire_g(1)
  fire_idx(3, 3)
  for b in range(NBUF):
    i = b
    wait_g(b)
    transpose(b)
    fire_out(b, i)
    fire_idx(b, i + NBUF)
    nb = (b + 2) % NBUF
    wait_idx(nb, i + 2)
    fire_g(nb)

  @pl.loop(NBUF, n_items - NBUF, step=NBUF)
  def _steady(g0):
    for b in range(NBUF):
      i = g0 + b
      wait_g(b)
      wait_out(b, i - NBUF)
      transpose(b)
      fire_out(b, i)
      fire_idx(b, i + NBUF)
      nb = (b + 2) % NBUF
      wait_idx(nb, i + 2)
      fire_g(nb)

  for b in range(NBUF):
    i = n_items - NBUF + b
    wait_g(b)
    wait_out(b, i - NBUF)
    transpose(b)
    fire_out(b, i)
    if b < 2:
      nb = (b + 2) % NBUF
      wait_idx(nb, i + 2)
      fire_g(nb)
  for b in range(NBUF):
    wait_out(b, n_items - NBUF + b)


def kernel(x, table):
  batch, seq = x.shape
  n_chunks = batch // CHUNK
  total_items = seq * n_chunks
  assert total_items % NUM_WORKERS == 0
  n_items = total_items // NUM_WORKERS
  assert n_items % NBUF == 0 and n_items >= 3 * NBUF

  xt = jnp.transpose(x.astype(jnp.int32))

  mesh = plsc.VectorSubcoreMesh(core_axis_name="c", subcore_axis_name="s")
  run = pl.kernel(
      functools.partial(_embed_kernel, seq, n_chunks, n_items),
      out_type=jax.ShapeDtypeStruct((seq, DIM, batch), jnp.float32),
      mesh=mesh,
      scratch_types=[
          pltpu.VMEM((NBUF, CHUNK), jnp.int32),
          pltpu.VMEM((NBUF, CHUNK, DIM), jnp.float32),
          pltpu.VMEM((NBUF, DIM, CHUNK), jnp.float32),
          [pltpu.SemaphoreType.DMA] * NBUF,
          [pltpu.SemaphoreType.DMA] * NBUF,
          [pltpu.SemaphoreType.DMA] * NBUF,
      ],
      compiler_params=pltpu.CompilerParams(
          use_tc_tiling_on_sc=False, needs_layout_passes=False),
  )
  out = run(table, xt)
  return jnp.transpose(out, (2, 0, 1))


# speedup vs baseline: 1.5972x; 1.2076x over previous
import functools

import jax
import jax.numpy as jnp
from jax import lax
from jax.experimental import pallas as pl
from jax.experimental.pallas import tpu as pltpu
from jax.experimental.pallas import tpu_sc as plsc

NUM_CORES = 2
NUM_SUBCORES = 16
NUM_WORKERS = NUM_CORES * NUM_SUBCORES

DIM = 64
PAIR = 2 * DIM
CHUNK = 128
NBUF = 4


def _embed_kernel(seq, n_chunks, n_items, table_hbm, xt_hbm, out_hbm,
                  idx_v, pid_v, rows_v, t_v, isems, gsems, osems):
  wid = lax.axis_index("s") * NUM_CORES + lax.axis_index("c")
  item0 = wid * n_items
  iota = lax.iota(jnp.int32, 16)

  def sb(i):
    t = item0 + i
    return t // n_chunks, (t % n_chunks) * CHUNK

  def fire_idx(b, i):
    s, b0 = sb(i)
    pltpu.async_copy(xt_hbm.at[s, pl.ds(b0, CHUNK)], idx_v.at[b], isems[b])

  def wait_idx(b, i):
    s, b0 = sb(i)
    pltpu.make_async_copy(
        xt_hbm.at[s, pl.ds(b0, CHUNK)], idx_v.at[b], isems[b]).wait()

  def make_pids(b):
    @plsc.parallel_loop(0, CHUNK // 16, unroll=4)
    def _p(j):
      pid_v[b, pl.ds(j * 16, 16)] = (
          lax.shift_right_logical(idx_v[b, pl.ds(j * 16, 16)], 1))

  def fire_g(b):
    pltpu.async_copy(table_hbm.at[pid_v.at[b]], rows_v.at[b], gsems[b])

  def wait_g(b):
    pltpu.make_async_copy(
        table_hbm.at[pid_v.at[b]], rows_v.at[b], gsems[b]).wait()

  def transpose(b):
    @pl.loop(0, CHUNK // 16)
    def _grp(j):
      row_idx = j * 16 + iota
      toks = idx_v[b, pl.ds(j * 16, 16)]
      colbase = lax.shift_left(
          lax.bitwise_and(toks, jnp.full((16,), 1, jnp.int32)), 6)

      @plsc.parallel_loop(0, DIM, unroll=16)
      def _t(d):
        vec = plsc.load_gather(rows_v.at[b], [row_idx, colbase + d])
        t_v[b, d, pl.ds(j * 16, 16)] = vec

  def fire_out(b, i):
    s, b0 = sb(i)
    pltpu.async_copy(t_v.at[b], out_hbm.at[s, :, pl.ds(b0, CHUNK)], osems[b])

  def wait_out(b, i):
    s, b0 = sb(i)
    pltpu.make_async_copy(
        t_v.at[b], out_hbm.at[s, :, pl.ds(b0, CHUNK)], osems[b]).wait()

  def prep_and_fire(b, i):
    wait_idx(b, i)
    make_pids(b)
    fire_g(b)

  fire_idx(0, 0)
  fire_idx(1, 1)
  prep_and_fire(0, 0)
  fire_idx(2, 2)
  prep_and_fire(1, 1)
  fire_idx(3, 3)
  for b in range(NBUF):
    i = b
    wait_g(b)
    transpose(b)
    fire_out(b, i)
    fire_idx(b, i + NBUF)
    prep_and_fire((b + 2) % NBUF, i + 2)

  @pl.loop(NBUF, n_items - NBUF, step=NBUF)
  def _steady(g0):
    for b in range(NBUF):
      i = g0 + b
      wait_g(b)
      wait_out(b, i - NBUF)
      transpose(b)
      fire_out(b, i)
      fire_idx(b, i + NBUF)
      prep_and_fire((b + 2) % NBUF, i + 2)

  for b in range(NBUF):
    i = n_items - NBUF + b
    wait_g(b)
    wait_out(b, i - NBUF)
    transpose(b)
    fire_out(b, i)
    if b < 2:
      prep_and_fire((b + 2) % NBUF, i + 2)
  for b in range(NBUF):
    wait_out(b, n_items - NBUF + b)


def kernel(x, table):
  batch, seq = x.shape
  vocab = table.shape[0]
  n_chunks = batch // CHUNK
  total_items = seq * n_chunks
  assert total_items % NUM_WORKERS == 0
  n_items = total_items // NUM_WORKERS
  assert n_items % NBUF == 0 and n_items >= 3 * NBUF

  xt = jnp.transpose(x.astype(jnp.int32))
  tpair = table.reshape(vocab // 2, PAIR)

  mesh = plsc.VectorSubcoreMesh(core_axis_name="c", subcore_axis_name="s")
  run = pl.kernel(
      functools.partial(_embed_kernel, seq, n_chunks, n_items),
      out_type=jax.ShapeDtypeStruct((seq, DIM, batch), jnp.float32),
      mesh=mesh,
      scratch_types=[
          pltpu.VMEM((NBUF, CHUNK), jnp.int32),
          pltpu.VMEM((NBUF, CHUNK), jnp.int32),
          pltpu.VMEM((NBUF, CHUNK, PAIR), jnp.float32),
          pltpu.VMEM((NBUF, DIM, CHUNK), jnp.float32),
          [pltpu.SemaphoreType.DMA] * NBUF,
          [pltpu.SemaphoreType.DMA] * NBUF,
          [pltpu.SemaphoreType.DMA] * NBUF,
      ],
      compiler_params=pltpu.CompilerParams(
          use_tc_tiling_on_sc=True, needs_layout_passes=False),
  )
  out = run(tpair, xt)
  return jnp.transpose(out, (2, 0, 1))


# speedup vs baseline: 1.6025x; 1.0033x over previous
import functools

import jax
import jax.numpy as jnp
from jax import lax
from jax.experimental import pallas as pl
from jax.experimental.pallas import tpu as pltpu
from jax.experimental.pallas import tpu_sc as plsc

NUM_CORES = 2
NUM_SUBCORES = 16
NUM_WORKERS = NUM_CORES * NUM_SUBCORES

DIM = 64
PAIR = 2 * DIM
CHUNK = 128
NBUF = 4


def _embed_kernel(seq, n_chunks, n_items, table_hbm, xt_hbm, out_hbm,
                  idx_v, pid_v, rows_v, t_v, isems, gsems, osems):
  wid = lax.axis_index("s") * NUM_CORES + lax.axis_index("c")
  item0 = wid * n_items
  iota = lax.iota(jnp.int32, 16)

  def sb(i):
    t = item0 + i
    return t // n_chunks, (t % n_chunks) * CHUNK

  def fire_idx(b, i):
    s, b0 = sb(i)
    pltpu.async_copy(xt_hbm.at[s, pl.ds(b0, CHUNK)], idx_v.at[b], isems[b])

  def wait_idx(b, i):
    s, b0 = sb(i)
    pltpu.make_async_copy(
        xt_hbm.at[s, pl.ds(b0, CHUNK)], idx_v.at[b], isems[b]).wait()

  def make_pids(b):
    @plsc.parallel_loop(0, CHUNK // 16, unroll=4)
    def _p(j):
      pid_v[b, pl.ds(j * 16, 16)] = (
          lax.shift_right_logical(idx_v[b, pl.ds(j * 16, 16)], 1))

  def fire_g(b):
    pltpu.async_copy(table_hbm.at[pid_v.at[b]], rows_v.at[b], gsems[b])

  def wait_g(b):
    pltpu.make_async_copy(
        table_hbm.at[pid_v.at[b]], rows_v.at[b], gsems[b]).wait()

  def transpose(b):
    @pl.loop(0, CHUNK // 16)
    def _grp(j):
      row_idx = j * 16 + iota
      toks = idx_v[b, pl.ds(j * 16, 16)]
      base = lax.shift_left(row_idx, 7) + lax.shift_left(
          lax.bitwise_and(toks, jnp.full((16,), 1, jnp.int32)), 6)

      zeros = jnp.full((16,), 0, jnp.int32)

      @plsc.parallel_loop(0, DIM, unroll=16, carry=base)
      def _t(d, fidx):
        vec = plsc.load_gather(rows_v.at[b], [zeros, fidx])
        t_v[b, d, pl.ds(j * 16, 16)] = vec
        return fidx + 1

  def fire_out(b, i):
    s, b0 = sb(i)
    pltpu.async_copy(t_v.at[b], out_hbm.at[s, :, pl.ds(b0, CHUNK)], osems[b])

  def wait_out(b, i):
    s, b0 = sb(i)
    pltpu.make_async_copy(
        t_v.at[b], out_hbm.at[s, :, pl.ds(b0, CHUNK)], osems[b]).wait()

  def prep_and_fire(b, i):
    wait_idx(b, i)
    make_pids(b)
    fire_g(b)

  fire_idx(0, 0)
  fire_idx(1, 1)
  prep_and_fire(0, 0)
  fire_idx(2, 2)
  prep_and_fire(1, 1)
  fire_idx(3, 3)
  for b in range(NBUF):
    i = b
    wait_g(b)
    transpose(b)
    fire_out(b, i)
    fire_idx(b, i + NBUF)
    prep_and_fire((b + 2) % NBUF, i + 2)

  @pl.loop(NBUF, n_items - NBUF, step=NBUF)
  def _steady(g0):
    for b in range(NBUF):
      i = g0 + b
      wait_g(b)
      wait_out(b, i - NBUF)
      transpose(b)
      fire_out(b, i)
      fire_idx(b, i + NBUF)
      prep_and_fire((b + 2) % NBUF, i + 2)

  for b in range(NBUF):
    i = n_items - NBUF + b
    wait_g(b)
    wait_out(b, i - NBUF)
    transpose(b)
    fire_out(b, i)
    if b < 2:
      prep_and_fire((b + 2) % NBUF, i + 2)
  for b in range(NBUF):
    wait_out(b, n_items - NBUF + b)


def kernel(x, table):
  batch, seq = x.shape
  vocab = table.shape[0]
  n_chunks = batch // CHUNK
  total_items = seq * n_chunks
  assert total_items % NUM_WORKERS == 0
  n_items = total_items // NUM_WORKERS
  assert n_items % NBUF == 0 and n_items >= 3 * NBUF

  xt = jnp.transpose(x.astype(jnp.int32))
  tpair = table.reshape(vocab // 2, PAIR)

  mesh = plsc.VectorSubcoreMesh(core_axis_name="c", subcore_axis_name="s")
  run = pl.kernel(
      functools.partial(_embed_kernel, seq, n_chunks, n_items),
      out_type=jax.ShapeDtypeStruct((seq, DIM, batch), jnp.float32),
      mesh=mesh,
      scratch_types=[
          pltpu.VMEM((NBUF, CHUNK), jnp.int32),
          pltpu.VMEM((NBUF, CHUNK), jnp.int32),
          pltpu.VMEM((NBUF, CHUNK, PAIR), jnp.float32),
          pltpu.VMEM((NBUF, DIM, CHUNK), jnp.float32),
          [pltpu.SemaphoreType.DMA] * NBUF,
          [pltpu.SemaphoreType.DMA] * NBUF,
          [pltpu.SemaphoreType.DMA] * NBUF,
      ],
      compiler_params=pltpu.CompilerParams(
          use_tc_tiling_on_sc=True, needs_layout_passes=False),
  )
  out = run(tpair, xt)
  return jnp.transpose(out, (2, 0, 1))


# speedup vs baseline: 1.6919x; 1.0558x over previous
import functools

import jax
import jax.numpy as jnp
from jax import lax
from jax.experimental import pallas as pl
from jax.experimental.pallas import tpu as pltpu
from jax.experimental.pallas import tpu_sc as plsc

NUM_CORES = 2
NUM_SUBCORES = 16
NUM_WORKERS = NUM_CORES * NUM_SUBCORES

DIM = 64
PAIR = 2 * DIM
CHUNK = 128
NBUF = 4


def _embed_kernel(seq, n_chunks, n_items, table_hbm, xt_hbm, out_hbm,
                  idx_v, rows_v, t_v, isems, gsems, osems):
  wid = lax.axis_index("s") * NUM_CORES + lax.axis_index("c")
  item0 = wid * n_items
  iota = lax.iota(jnp.int32, 16)

  def sb(i):
    t = item0 + i
    return t // n_chunks, (t % n_chunks) * CHUNK

  def fire_idx(b, i):
    s, b0 = sb(i)
    pltpu.async_copy(xt_hbm.at[s, pl.ds(b0, CHUNK)], idx_v.at[b], isems[b])

  def wait_idx(b, i):
    s, b0 = sb(i)
    pltpu.make_async_copy(
        xt_hbm.at[s, pl.ds(b0, CHUNK)], idx_v.at[b], isems[b]).wait()

  def fire_g(b):
    pltpu.async_copy(table_hbm.at[idx_v.at[b]], rows_v.at[b], gsems[b])

  def wait_g(b):
    pltpu.make_async_copy(
        table_hbm.at[idx_v.at[b]], rows_v.at[b], gsems[b]).wait()

  def transpose(b):
    @pl.loop(0, CHUNK // 16)
    def _grp(j):
      row_idx = j * 16 + iota
      base = lax.shift_left(row_idx, 7)

      zeros = jnp.full((16,), 0, jnp.int32)

      @plsc.parallel_loop(0, DIM, unroll=16, carry=base)
      def _t(d, fidx):
        vec = plsc.load_gather(rows_v.at[b], [zeros, fidx])
        t_v[b, d, pl.ds(j * 16, 16)] = vec
        return fidx + 1

  def fire_out(b, i):
    s, b0 = sb(i)
    pltpu.async_copy(t_v.at[b], out_hbm.at[s, :, pl.ds(b0, CHUNK)], osems[b])

  def wait_out(b, i):
    s, b0 = sb(i)
    pltpu.make_async_copy(
        t_v.at[b], out_hbm.at[s, :, pl.ds(b0, CHUNK)], osems[b]).wait()

  def prep_and_fire(b, i):
    wait_idx(b, i)
    fire_g(b)

  fire_idx(0, 0)
  fire_idx(1, 1)
  prep_and_fire(0, 0)
  fire_idx(2, 2)
  prep_and_fire(1, 1)
  fire_idx(3, 3)
  for b in range(NBUF):
    i = b
    wait_g(b)
    transpose(b)
    fire_out(b, i)
    fire_idx(b, i + NBUF)
    prep_and_fire((b + 2) % NBUF, i + 2)

  @pl.loop(NBUF, n_items - NBUF, step=NBUF)
  def _steady(g0):
    for b in range(NBUF):
      i = g0 + b
      wait_g(b)
      wait_out(b, i - NBUF)
      transpose(b)
      fire_out(b, i)
      fire_idx(b, i + NBUF)
      prep_and_fire((b + 2) % NBUF, i + 2)

  for b in range(NBUF):
    i = n_items - NBUF + b
    wait_g(b)
    wait_out(b, i - NBUF)
    transpose(b)
    fire_out(b, i)
    if b < 2:
      prep_and_fire((b + 2) % NBUF, i + 2)
  for b in range(NBUF):
    wait_out(b, n_items - NBUF + b)


def kernel(x, table):
  batch, seq = x.shape
  vocab = table.shape[0]
  n_chunks = batch // CHUNK
  total_items = seq * n_chunks
  assert total_items % NUM_WORKERS == 0
  n_items = total_items // NUM_WORKERS
  assert n_items % NBUF == 0 and n_items >= 3 * NBUF

  xt = jnp.transpose(x.astype(jnp.int32))
  tpad = jnp.pad(table, ((0, 0), (0, DIM)))

  mesh = plsc.VectorSubcoreMesh(core_axis_name="c", subcore_axis_name="s")
  run = pl.kernel(
      functools.partial(_embed_kernel, seq, n_chunks, n_items),
      out_type=jax.ShapeDtypeStruct((seq, DIM, batch), jnp.float32),
      mesh=mesh,
      scratch_types=[
          pltpu.VMEM((NBUF, CHUNK), jnp.int32),
          pltpu.VMEM((NBUF, CHUNK, PAIR), jnp.float32),
          pltpu.VMEM((NBUF, DIM, CHUNK), jnp.float32),
          [pltpu.SemaphoreType.DMA] * NBUF,
          [pltpu.SemaphoreType.DMA] * NBUF,
          [pltpu.SemaphoreType.DMA] * NBUF,
      ],
      compiler_params=pltpu.CompilerParams(
          use_tc_tiling_on_sc=True, needs_layout_passes=False),
  )
  out = run(tpad, xt)
  return jnp.transpose(out, (2, 0, 1))
